# baseline (speedup 1.0000x reference)
import functools

import jax
import jax.numpy as jnp
from jax import lax
from jax.experimental import pallas as pl
from jax.experimental.pallas import tpu as pltpu
from jax.experimental.pallas import tpu_sc as plsc

NC = 2
NS = 16
L = 16
BS = 1024
CC = 64
CAP = 96
REG = 2

_BCAST_DNUMS = lax.GatherDimensionNumbers(
    offset_dims=(), collapsed_slice_dims=(0,), start_index_map=(0,))


def _bcast_lane(v, j):
    idx = jnp.full((L, 1), j, jnp.int32)
    return lax.gather(v, idx, dimension_numbers=_BCAST_DNUMS, slice_sizes=(1,),
                      mode=lax.GatherScatterMode.PROMISE_IN_BOUNDS)


def _sc_body(n_out, n_in, batch, n_blocks,
             x_hbm, rows_hbm, cols_hbm, scal_hbm, zf_hbm, zi_hbm, out_hbm,
             slab, rbuf, cbuf, sbuf, pk, bscal, bpos, xchunk,
             msem0, msem1, xsem0, xsem1):
    c = lax.axis_index("c")
    s = lax.axis_index("s")
    window = n_out // NS
    w0 = s * window
    cr = n_in // CC
    cshift = 0
    while (1 << cshift) * CC < n_in:
        cshift += 1
    lanes = lax.iota(jnp.int32, L)

    pltpu.sync_copy(zf_hbm, slab)
    pltpu.sync_copy(zf_hbm.at[pl.ds(0, REG * CC * CAP)], bscal)
    pltpu.sync_copy(zi_hbm, pk)
    for t in range(REG * CC // L):
        bpos[pl.ds(t * L, L)] = (lanes + t * L) * CAP

    def meta_start(nb, slot, sem):
        pltpu.async_copy(rows_hbm.at[c, nb], rbuf.at[slot], sem)
        pltpu.async_copy(cols_hbm.at[c, nb], cbuf.at[slot], sem)
        pltpu.async_copy(scal_hbm.at[c, nb], sbuf.at[slot], sem)

    def meta_wait(nb, slot, sem):
        pltpu.make_async_copy(rows_hbm.at[c, nb], rbuf.at[slot], sem).wait()
        pltpu.make_async_copy(cols_hbm.at[c, nb], cbuf.at[slot], sem).wait()
        pltpu.make_async_copy(scal_hbm.at[c, nb], sbuf.at[slot], sem).wait()

    meta_start(0, 0, msem0)

    def block(nb, carry0):
        slot = lax.rem(nb, 2)

        @pl.when((nb + 1 < n_blocks) & (slot == 0))
        def _():
            meta_start(nb + 1, 1, msem1)

        @pl.when((nb + 1 < n_blocks) & (slot == 1))
        def _():
            meta_start(nb + 1, 0, msem0)

        @pl.when(slot == 0)
        def _():
            meta_wait(nb, 0, msem0)

        @pl.when(slot == 1)
        def _():
            meta_wait(nb, 1, msem1)

        def scan(i, carry1):
            for r in range(REG):
                o = i * (REG * L) + r * L
                rv = rbuf[slot, pl.ds(o, L)]
                cv = cbuf[slot, pl.ds(o, L)]
                sv = sbuf[slot, pl.ds(o, L)]
                lr = rv - w0
                m = (lr >= 0) & (lr < window)
                idx = lax.shift_right_logical(cv, cshift) + (r * CC)
                base = plsc.load_gather(bpos, [idx])
                cntv, lastm = plsc.scan_count(idx, mask=m)
                lim = idx * CAP + (CAP - 1)
                pos = jnp.minimum(base + cntv - 1, lim)
                pkv = lax.shift_left(cv & (cr - 1), 8) | lr
                plsc.store_scatter(pk, [pos], pkv, mask=m)
                plsc.store_scatter(bscal, [pos], sv, mask=m)
                plsc.store_scatter(bpos, [idx], jnp.minimum(pos + 1, lim),
                                   mask=m & lastm)
            return carry1

        lax.fori_loop(0, BS // (REG * L), scan, 0, unroll=2)
        return carry0

    lax.fori_loop(0, n_blocks, block, 0)

    def x_start(cc, slot, sem):
        pltpu.async_copy(x_hbm.at[pl.ds(cc * (cr * batch), cr * batch)],
                         xchunk.at[slot], sem)

    def x_wait(cc, slot, sem):
        pltpu.make_async_copy(x_hbm.at[pl.ds(cc * (cr * batch), cr * batch)],
                              xchunk.at[slot], sem).wait()

    x_start(0, 0, xsem0)

    def chunk(cc, carry0):
        xslot = lax.rem(cc, 2)

        @pl.when((cc + 1 < CC) & (xslot == 0))
        def _():
            x_start(cc + 1, 1, xsem1)

        @pl.when((cc + 1 < CC) & (xslot == 1))
        def _():
            x_start(cc + 1, 0, xsem0)

        @pl.when(xslot == 0)
        def _():
            x_wait(cc, 0, xsem0)

        @pl.when(xslot == 1)
        def _():
            x_wait(cc, 1, xsem1)

        cntv = plsc.load_gather(bpos, [jnp.zeros((L,), jnp.int32) + cc])
        n_e = (jnp.max(cntv) - cc * CAP + (L - 1)) // L
        cntv2 = plsc.load_gather(bpos,
                                 [jnp.zeros((L,), jnp.int32) + (CC + cc)])
        n_e2 = (jnp.max(cntv2) - (CC + cc) * CAP + (L - 1)) // L

        def entry2(e, carry1):
            off = (CC + cc) * CAP + e * L
            pkv = pk[pl.ds(off, L)]
            sv = bscal[pl.ds(off, L)]
            rloc = (pkv & 255) * batch
            cloc = lax.shift_right_logical(pkv, 8) * batch
            nq = batch // L
            for j in range(L):
                c_j = cloc[j]
                r_j = rloc[j]
                s_j = sv[j]
                vals = [xchunk[xslot, pl.ds(c_j + q * L, L)]
                        for q in range(nq)]
                vals = [v * s_j for v in vals]
                for q in range(nq):
                    plsc.addupdate(slab.at[pl.ds(r_j + q * L, L)], vals[q])
            return carry1

        lax.fori_loop(0, n_e2, entry2, 0)

        def entry(e, carry1):
            off = cc * CAP + e * L
            pkv = pk[pl.ds(off, L)]
            sv = bscal[pl.ds(off, L)]
            rloc = (pkv & 255) * batch
            cloc = lax.shift_right_logical(pkv, 8) * batch
            nq = batch // L
            for j in range(L):
                c_j = cloc[j]
                r_j = rloc[j]
                s_j = sv[j]
                vals = [xchunk[xslot, pl.ds(c_j + q * L, L)]
                        for q in range(nq)]
                vals = [v * s_j for v in vals]
                for q in range(nq):
                    plsc.addupdate(slab.at[pl.ds(r_j + q * L, L)], vals[q])
            return carry1

        lax.fori_loop(0, n_e, entry, 0)
        return carry0

    lax.fori_loop(0, CC, chunk, 0)

    pltpu.sync_copy(slab, out_hbm.at[c, pl.ds(w0 * batch, window * batch)])


def _sc_spmm(x_t, rows_p, cols_p, scal_p, zf, zi, n_out, n_in, batch,
             n_blocks):
    mesh = plsc.VectorSubcoreMesh(core_axis_name="c", subcore_axis_name="s",
                                  num_cores=NC, num_subcores=NS)
    window = n_out // NS
    cr = n_in // CC
    body = functools.partial(_sc_body, n_out, n_in, batch, n_blocks)
    return pl.kernel(
        body,
        out_type=jax.ShapeDtypeStruct((NC, n_out * batch), jnp.float32),
        mesh=mesh,
        compiler_params=pltpu.CompilerParams(needs_layout_passes=False),
        scratch_types=[
            pltpu.VMEM((window * batch,), jnp.float32),
            pltpu.VMEM((2, BS), jnp.int32),
            pltpu.VMEM((2, BS), jnp.int32),
            pltpu.VMEM((2, BS), jnp.float32),
            pltpu.VMEM((REG * CC * CAP,), jnp.int32),
            pltpu.VMEM((REG * CC * CAP,), jnp.float32),
            pltpu.VMEM((REG * CC,), jnp.int32),
            pltpu.VMEM((2, cr * batch), jnp.float32),
            pltpu.SemaphoreType.DMA,
            pltpu.SemaphoreType.DMA,
            pltpu.SemaphoreType.DMA,
            pltpu.SemaphoreType.DMA,
        ],
    )(x_t, rows_p, cols_p, scal_p, zf, zi)


def _tc_finish_body(y_ref, off_ref, o_ref):
    y = y_ref[0] + y_ref[1] + off_ref[...]
    o_ref[...] = y.T


def kernel(inputs, indices, scaling, offset):
    batch, n_in = inputs.shape
    n_out = offset.shape[0]
    nnz = scaling.shape[0]
    half = -(-nnz // (NC * BS)) * BS
    n_blocks = half // BS
    pad = NC * half - nnz

    rows = jnp.concatenate(
        [indices[:, 0], jnp.zeros((pad,), jnp.int32)]).reshape(NC, n_blocks, BS)
    cols = jnp.concatenate(
        [indices[:, 1], jnp.zeros((pad,), jnp.int32)]).reshape(NC, n_blocks, BS)
    scal = jnp.concatenate(
        [scaling, jnp.zeros((pad,), jnp.float32)]).reshape(NC, n_blocks, BS)
    x_t = inputs.T.reshape(-1)
    window = n_out // NS
    zf = jnp.zeros((window * batch,), jnp.float32)
    zi = jnp.zeros((REG * CC * CAP,), jnp.int32)

    y2 = _sc_spmm(x_t, rows, cols, scal, zf, zi, n_out, n_in, batch, n_blocks)
    y2 = y2.reshape(NC, n_out, batch)

    return pl.pallas_call(
        _tc_finish_body,
        out_shape=jax.ShapeDtypeStruct((batch, n_out), jnp.float32),
    )(y2, offset)


# speedup vs baseline: 1.0558x; 1.0558x over previous
import functools

import jax
import jax.numpy as jnp
from jax import lax
from jax.experimental import pallas as pl
from jax.experimental.pallas import tpu as pltpu
from jax.experimental.pallas import tpu_sc as plsc

NC = 2
NS = 16
L = 16
BS = 2048
CC = 64
CAP = 158

_BCAST_DNUMS = lax.GatherDimensionNumbers(
    offset_dims=(), collapsed_slice_dims=(0,), start_index_map=(0,))


def _bcast_lane(v, j):
    idx = jnp.full((L, 1), j, jnp.int32)
    return lax.gather(v, idx, dimension_numbers=_BCAST_DNUMS, slice_sizes=(1,),
                      mode=lax.GatherScatterMode.PROMISE_IN_BOUNDS)


def _sc_body(n_out, n_in, batch, n_blocks,
             x_hbm, rows_hbm, cols_hbm, scal_hbm, zf_hbm, zi_hbm, out_hbm,
             slab, rbuf, cbuf, sbuf, pk, bscal, bpos, xchunk,
             msem0, msem1, xsem0, xsem1):
    c = lax.axis_index("c")
    s = lax.axis_index("s")
    window = n_out // NS
    w0 = s * window
    cr = n_in // CC
    cshift = 0
    while (1 << cshift) * CC < n_in:
        cshift += 1
    lanes = lax.iota(jnp.int32, L)

    pltpu.sync_copy(zf_hbm, slab)
    pltpu.sync_copy(zf_hbm.at[pl.ds(0, CC * CAP)], bscal)
    pltpu.sync_copy(zi_hbm, pk)
    for t in range(CC // L):
        bpos[pl.ds(t * L, L)] = (lanes + t * L) * CAP

    def meta_start(nb, slot, sem):
        pltpu.async_copy(rows_hbm.at[c, nb], rbuf.at[slot], sem)
        pltpu.async_copy(cols_hbm.at[c, nb], cbuf.at[slot], sem)
        pltpu.async_copy(scal_hbm.at[c, nb], sbuf.at[slot], sem)

    def meta_wait(nb, slot, sem):
        pltpu.make_async_copy(rows_hbm.at[c, nb], rbuf.at[slot], sem).wait()
        pltpu.make_async_copy(cols_hbm.at[c, nb], cbuf.at[slot], sem).wait()
        pltpu.make_async_copy(scal_hbm.at[c, nb], sbuf.at[slot], sem).wait()

    meta_start(0, 0, msem0)

    def block(nb, carry0):
        slot = lax.rem(nb, 2)

        @pl.when((nb + 1 < n_blocks) & (slot == 0))
        def _():
            meta_start(nb + 1, 1, msem1)

        @pl.when((nb + 1 < n_blocks) & (slot == 1))
        def _():
            meta_start(nb + 1, 0, msem0)

        @pl.when(slot == 0)
        def _():
            meta_wait(nb, 0, msem0)

        @pl.when(slot == 1)
        def _():
            meta_wait(nb, 1, msem1)

        def scan(i, carry1):
            rv = rbuf[slot, pl.ds(i * L, L)]
            cv = cbuf[slot, pl.ds(i * L, L)]
            sv = sbuf[slot, pl.ds(i * L, L)]
            lr = rv - w0
            m = (lr >= 0) & (lr < window)
            ccv = lax.shift_right_logical(cv, cshift)
            base = plsc.load_gather(bpos, [ccv])
            cntv, lastm = plsc.scan_count(ccv, mask=m)
            lim = ccv * CAP + (CAP - 1)
            pos = jnp.minimum(base + cntv - 1, lim)
            pkv = lax.shift_left(cv & (cr - 1), 8) | lr
            plsc.store_scatter(pk, [pos], pkv, mask=m)
            plsc.store_scatter(bscal, [pos], sv, mask=m)
            plsc.store_scatter(bpos, [ccv], jnp.minimum(pos + 1, lim),
                               mask=m & lastm)
            return carry1

        lax.fori_loop(0, BS // L, scan, 0, unroll=2)
        return carry0

    lax.fori_loop(0, n_blocks, block, 0)

    def x_start(cc, slot, sem):
        pltpu.async_copy(x_hbm.at[pl.ds(cc * (cr * batch), cr * batch)],
                         xchunk.at[slot], sem)

    def x_wait(cc, slot, sem):
        pltpu.make_async_copy(x_hbm.at[pl.ds(cc * (cr * batch), cr * batch)],
                              xchunk.at[slot], sem).wait()

    x_start(0, 0, xsem0)

    def chunk(cc, carry0):
        xslot = lax.rem(cc, 2)

        @pl.when((cc + 1 < CC) & (xslot == 0))
        def _():
            x_start(cc + 1, 1, xsem1)

        @pl.when((cc + 1 < CC) & (xslot == 1))
        def _():
            x_start(cc + 1, 0, xsem0)

        @pl.when(xslot == 0)
        def _():
            x_wait(cc, 0, xsem0)

        @pl.when(xslot == 1)
        def _():
            x_wait(cc, 1, xsem1)

        cntv = plsc.load_gather(bpos, [jnp.zeros((L,), jnp.int32) + cc])
        n_e = (jnp.max(cntv) - cc * CAP + (L - 1)) // L

        def entry(e, carry1):
            off = cc * CAP + e * L
            pkv = pk[pl.ds(off, L)]
            sv = bscal[pl.ds(off, L)]
            rloc = (pkv & 255) * batch
            cloc = lax.shift_right_logical(pkv, 8) * batch
            nq = batch // L
            for j in range(L):
                c_j = cloc[j]
                r_j = rloc[j]
                s_j = sv[j]
                vals = [xchunk[xslot, pl.ds(c_j + q * L, L)]
                        for q in range(nq)]
                vals = [v * s_j for v in vals]
                for q in range(nq):
                    plsc.addupdate(slab.at[pl.ds(r_j + q * L, L)], vals[q])
            return carry1

        lax.fori_loop(0, n_e, entry, 0)
        return carry0

    lax.fori_loop(0, CC, chunk, 0)

    pltpu.sync_copy(slab, out_hbm.at[c, pl.ds(w0 * batch, window * batch)])


def _sc_spmm(x_t, rows_p, cols_p, scal_p, zf, zi, n_out, n_in, batch,
             n_blocks):
    mesh = plsc.VectorSubcoreMesh(core_axis_name="c", subcore_axis_name="s",
                                  num_cores=NC, num_subcores=NS)
    window = n_out // NS
    cr = n_in // CC
    body = functools.partial(_sc_body, n_out, n_in, batch, n_blocks)
    return pl.kernel(
        body,
        out_type=jax.ShapeDtypeStruct((NC, n_out * batch), jnp.float32),
        mesh=mesh,
        compiler_params=pltpu.CompilerParams(needs_layout_passes=False),
        scratch_types=[
            pltpu.VMEM((window * batch,), jnp.float32),
            pltpu.VMEM((2, BS), jnp.int32),
            pltpu.VMEM((2, BS), jnp.int32),
            pltpu.VMEM((2, BS), jnp.float32),
            pltpu.VMEM((CC * CAP,), jnp.int32),
            pltpu.VMEM((CC * CAP,), jnp.float32),
            pltpu.VMEM((CC,), jnp.int32),
            pltpu.VMEM((2, cr * batch), jnp.float32),
            pltpu.SemaphoreType.DMA,
            pltpu.SemaphoreType.DMA,
            pltpu.SemaphoreType.DMA,
            pltpu.SemaphoreType.DMA,
        ],
    )(x_t, rows_p, cols_p, scal_p, zf, zi)


def _tc_finish_body(y_ref, off_ref, o_ref):
    y = y_ref[0] + y_ref[1] + off_ref[...]
    o_ref[...] = y.T


def kernel(inputs, indices, scaling, offset):
    batch, n_in = inputs.shape
    n_out = offset.shape[0]
    nnz = scaling.shape[0]
    half = -(-nnz // (NC * BS)) * BS
    n_blocks = half // BS
    pad = NC * half - nnz

    rows = jnp.concatenate(
        [indices[:, 0], jnp.zeros((pad,), jnp.int32)]).reshape(NC, n_blocks, BS)
    cols = jnp.concatenate(
        [indices[:, 1], jnp.zeros((pad,), jnp.int32)]).reshape(NC, n_blocks, BS)
    scal = jnp.concatenate(
        [scaling, jnp.zeros((pad,), jnp.float32)]).reshape(NC, n_blocks, BS)
    x_t = inputs.T.reshape(-1)
    window = n_out // NS
    zf = jnp.zeros((window * batch,), jnp.float32)
    zi = jnp.zeros((CC * CAP,), jnp.int32)

    y2 = _sc_spmm(x_t, rows, cols, scal, zf, zi, n_out, n_in, batch, n_blocks)
    y2 = y2.reshape(NC, n_out, batch)

    return pl.pallas_call(
        _tc_finish_body,
        out_shape=jax.ShapeDtypeStruct((batch, n_out), jnp.float32),
    )(y2, offset)
